# baseline (speedup 1.0000x reference)
import math

import jax
import jax.numpy as jnp
from jax.experimental import pallas as pl
from jax.experimental.pallas import tpu as pltpu

_K = 64
_EXP = 0.999
_LN_EXP = math.log(_EXP)
_NEG_BIG = -3.0e38
_POS_BIG = 3.0e38

_NBUF = 8
_BM = 128
_WIDE = 4096
_NARROW = 2048

_ALPHA0 = 0
_GI0 = 8192
_GH0 = 8192 + 6144
_PART0 = 8192 + 6144 + 6144
_MTOT = 8192 + 6144 + 6144 + 4096


def _dot_nt(w, x):
    return jax.lax.dot_general(w, x, (((1,), (1,)), ((), ())),
                               preferred_element_type=jnp.float32)


def _manual_mv(slot_ref, xb):
    n = xb.shape[1]
    nch = n // 128
    cols = []
    for g in range(16):
        prod = slot_ref[pl.ds(g * 8, 8), :] * xb
        parts = [prod[:, c * 128:(c + 1) * 128] for c in range(nch)]
        while len(parts) > 1:
            nxt = [parts[i] + parts[i + 1] for i in range(0, len(parts) - 1, 2)]
            if len(parts) % 2:
                nxt.append(parts[-1])
            parts = nxt
        cols.append(jnp.sum(parts[0], axis=1, keepdims=True))
    return jnp.concatenate(cols, axis=0)


def _mega_body(fs_ref, wih_ref, whh_ref, wout_ref, xw_ref, xcat_ref, b_ref,
               y_ref, ring_ref, sems):
    y_ref[...] = b_ref[...]
    xw = xw_ref[...]

    def wide_phase(src_ref, n_tiles, out_base):
        def issue(t, b):
            pltpu.make_async_copy(
                src_ref.at[pl.ds(t * _BM, _BM)], ring_ref.at[b],
                sems.at[b, 0]).start()

        def wait_compute(t, b):
            pltpu.make_async_copy(
                src_ref.at[pl.ds(t * _BM, _BM)], ring_ref.at[b],
                sems.at[b, 0]).wait()
            y = _manual_mv(ring_ref.at[b], xw)
            y_ref[pl.ds(out_base + t * _BM, _BM), :] += y

        for b in range(_NBUF):
            issue(b, b)
        n_groups = n_tiles // _NBUF

        def group(g, _):
            for b in range(_NBUF):
                t = g * _NBUF + b
                wait_compute(t, b)

                @pl.when(g < n_groups - 1)
                def _pref():
                    issue(t + _NBUF, b)
            return 0

        jax.lax.fori_loop(0, n_groups, group, 0)

    def narrow_phase(src_ref, n_tiles, tile_map):
        def issue(t, b, half):
            r0, c0, _, _ = tile_map(t)
            pltpu.make_async_copy(
                src_ref.at[pl.ds(r0, _BM), pl.ds(c0, _NARROW)],
                ring_ref.at[b, slice(None), pl.ds(half * _NARROW, _NARROW)],
                sems.at[b, half]).start()

        def wait_compute(t, b, half):
            r0, c0, xsel, o0 = tile_map(t)
            pltpu.make_async_copy(
                src_ref.at[pl.ds(r0, _BM), pl.ds(c0, _NARROW)],
                ring_ref.at[b, slice(None), pl.ds(half * _NARROW, _NARROW)],
                sems.at[b, half]).wait()
            xrow = xcat_ref[pl.ds(xsel * 8, 8), :]
            y = _manual_mv(ring_ref.at[b, slice(None), pl.ds(half * _NARROW, _NARROW)], xrow)
            y_ref[pl.ds(o0, _BM), :] += y

        for b in range(_NBUF):
            issue(2 * b, b, 0)
            issue(2 * b + 1, b, 1)
        n_groups = n_tiles // (2 * _NBUF)

        def group(g, _):
            for b in range(_NBUF):
                for half in range(2):
                    t = (g * _NBUF + b) * 2 + half
                    wait_compute(t, b, half)

                    @pl.when(g < n_groups - 1)
                    def _pref():
                        issue(t + 2 * _NBUF, b, half)
            return 0

        jax.lax.fori_loop(0, n_groups, group, 0)

    wide_phase(fs_ref, 64, _ALPHA0)
    wide_phase(wih_ref, 48, _GI0)

    def whh_map(t):
        return t * _BM, 0, 2, _GH0 + t * _BM

    narrow_phase(whh_ref, 48, whh_map)

    def wout_map(t):
        i = t // 3
        j = t - 3 * i
        c0 = jnp.where(j == 2, 6144, j * _NARROW)
        return i * _BM, c0, j, _PART0 + i * _BM

    narrow_phase(wout_ref, 96, wout_map)


def _mega_matvec(fs, W_ih, W_hh, W_out, feature, h, bias_cat):
    xw = jnp.broadcast_to(feature.reshape(1, _WIDE), (8, _WIDE))
    xcat = jnp.concatenate(
        [jnp.broadcast_to(feature.reshape(2, 1, _NARROW), (2, 8, _NARROW)).reshape(16, _NARROW),
         jnp.broadcast_to(h.reshape(1, _NARROW), (8, _NARROW)),
         jnp.zeros((8, _NARROW), jnp.float32)], axis=0)
    return pl.pallas_call(
        _mega_body,
        grid=(1,),
        in_specs=[
            pl.BlockSpec(memory_space=pltpu.HBM),
            pl.BlockSpec(memory_space=pltpu.HBM),
            pl.BlockSpec(memory_space=pltpu.HBM),
            pl.BlockSpec(memory_space=pltpu.HBM),
            pl.BlockSpec(memory_space=pltpu.VMEM),
            pl.BlockSpec(memory_space=pltpu.VMEM),
            pl.BlockSpec(memory_space=pltpu.VMEM),
        ],
        out_specs=pl.BlockSpec(memory_space=pltpu.VMEM),
        out_shape=jax.ShapeDtypeStruct((_MTOT, 1), jnp.float32),
        scratch_shapes=[
            pltpu.VMEM((_NBUF, _BM, _WIDE), jnp.float32),
            pltpu.SemaphoreType.DMA((_NBUF, 2)),
        ],
    )(fs, W_ih, W_hh, W_out, xw, xcat, bias_cat)


def _select_body(alpha_ref, elapsed_ref, hs_ref, attn_ref,
                 idx_ref, rows_ref, sem):
    alpha = alpha_ref[...]
    elapsed = elapsed_ref[...]
    rows_i = jax.lax.broadcasted_iota(jnp.int32, alpha.shape, 0)
    cols_i = jax.lax.broadcasted_iota(jnp.int32, alpha.shape, 1)
    flat_f = (rows_i * 1024 + cols_i).astype(jnp.float32)
    col64 = jax.lax.broadcasted_iota(jnp.int32, (1, _K), 1)

    def body(k, carry):
        masked, vals = carry
        m = jnp.max(masked)
        eq = masked == m
        idx_f = jnp.min(jnp.where(eq, flat_f, _POS_BIG))
        hit = flat_f == idx_f
        el = jnp.min(jnp.where(hit, elapsed, _POS_BIG))
        decayed = m * jnp.exp(_LN_EXP * el)
        vals = jnp.where(col64 == k, decayed, vals)
        idx_ref[k] = idx_f.astype(jnp.int32)
        masked = jnp.where(hit, _NEG_BIG, masked)
        return masked, vals

    _, vals = jax.lax.fori_loop(
        0, _K, body, (alpha, jnp.zeros((1, _K), jnp.float32)))

    vmax = jnp.max(vals)
    e = jnp.exp(vals - vmax)
    w = e / jnp.sum(e)

    for k in range(_K):
        pltpu.make_async_copy(
            hs_ref.at[pl.ds(idx_ref[k], 1)], rows_ref.at[pl.ds(k, 1)], sem
        ).start()
    for k in range(_K):
        pltpu.make_async_copy(
            hs_ref.at[pl.ds(idx_ref[k], 1)], rows_ref.at[pl.ds(k, 1)], sem
        ).wait()

    attn_ref[...] = jax.lax.dot_general(
        w, rows_ref[...], (((1,), (0,)), ((), ())),
        preferred_element_type=jnp.float32)


def _select_gather(alpha, elapsed, hs):
    h_dim = hs.shape[1]
    return pl.pallas_call(
        _select_body,
        in_specs=[
            pl.BlockSpec(memory_space=pltpu.VMEM),
            pl.BlockSpec(memory_space=pltpu.VMEM),
            pl.BlockSpec(memory_space=pltpu.HBM),
        ],
        out_specs=pl.BlockSpec(memory_space=pltpu.VMEM),
        out_shape=jax.ShapeDtypeStruct((1, h_dim), jnp.float32),
        scratch_shapes=[
            pltpu.SMEM((_K,), jnp.int32),
            pltpu.VMEM((_K, h_dim), jnp.float32),
            pltpu.SemaphoreType.DMA,
        ],
    )(alpha.reshape(8, 1024), elapsed.reshape(8, 1024), hs)


def _final_body(wmid_ref, attn_ref, partial_ref, gi_ref, gh_ref, h_ref,
                out_ref, hnew_ref):
    pred = partial_ref[...] + jax.lax.dot_general(
        attn_ref[...], wmid_ref[...], (((1,), (1,)), ((), ())),
        preferred_element_type=jnp.float32)
    m = jnp.max(pred)
    lse = jnp.log(jnp.sum(jnp.exp(pred - m))) + m
    out_ref[...] = pred - lse

    gi = gi_ref[...]
    gh = gh_ref[...]
    hdim = h_ref.shape[1]
    i_r = gi[:, :hdim]
    i_z = gi[:, hdim:2 * hdim]
    i_n = gi[:, 2 * hdim:]
    h_r = gh[:, :hdim]
    h_z = gh[:, hdim:2 * hdim]
    h_n = gh[:, 2 * hdim:]
    r = jax.nn.sigmoid(i_r + h_r)
    z = jax.nn.sigmoid(i_z + h_z)
    n = jnp.tanh(i_n + r * h_n)
    hnew_ref[...] = (1.0 - z) * n + z * h_ref[...]


def kernel(feature, time, fs, hs, ts, W_ih, W_hh, b_ih, b_hh, W_out, b_out):
    feature = feature.astype(jnp.float32)
    L, in_dim = fs.shape
    h_dim = hs.shape[1]
    out_dim = W_out.shape[0]
    h = hs[-1]

    elapsed = jnp.float32(time) - ts

    w_last = jax.lax.slice(W_out, (0, in_dim + 2 * h_dim),
                           (out_dim, in_dim + 2 * h_dim + 1))
    bias_cat = jnp.concatenate([
        jnp.zeros((L,), jnp.float32), b_ih, b_hh,
        b_out + float(_K) * w_last.reshape(-1),
    ]).reshape(_MTOT, 1)

    y = _mega_matvec(fs, W_ih, W_hh, W_out, feature, h, bias_cat)
    alpha = y[_ALPHA0:_ALPHA0 + L]
    gi = y[_GI0:_GI0 + 3 * h_dim]
    gh = y[_GH0:_GH0 + 3 * h_dim]
    partial = y[_PART0:_PART0 + out_dim]

    attn = _select_gather(alpha.reshape(-1), elapsed, hs)

    output, h_new = pl.pallas_call(
        _final_body,
        grid=(1,),
        in_specs=[
            pl.BlockSpec((out_dim, h_dim), lambda i: (0, 2)),
            pl.BlockSpec(memory_space=pltpu.VMEM),
            pl.BlockSpec(memory_space=pltpu.VMEM),
            pl.BlockSpec(memory_space=pltpu.VMEM),
            pl.BlockSpec(memory_space=pltpu.VMEM),
            pl.BlockSpec(memory_space=pltpu.VMEM),
        ],
        out_specs=[
            pl.BlockSpec(memory_space=pltpu.VMEM),
            pl.BlockSpec(memory_space=pltpu.VMEM),
        ],
        out_shape=[
            jax.ShapeDtypeStruct((1, out_dim), jnp.float32),
            jax.ShapeDtypeStruct((1, h_dim), jnp.float32),
        ],
    )(W_out, attn, partial.reshape(1, out_dim), gi.reshape(1, 3 * h_dim),
      gh.reshape(1, 3 * h_dim), h.reshape(1, h_dim))

    return output, h_new


# speedup vs baseline: 1.0465x; 1.0465x over previous
import math

import jax
import jax.numpy as jnp
from jax.experimental import pallas as pl
from jax.experimental.pallas import tpu as pltpu

_K = 64
_EXP = 0.999
_LN_EXP = math.log(_EXP)
_NEG_BIG = -3.0e38
_POS_BIG = 3.0e38

_NBUF = 8
_NWIDE = 6
_BM = 128
_WIDE = 4096
_NARROW = 2048

_ALPHA0 = 0
_GI0 = 8192
_GH0 = 8192 + 6144
_PART0 = 8192 + 6144 + 6144
_MTOT = 8192 + 6144 + 6144 + 4096


def _dot_nt(w, x):
    return jax.lax.dot_general(w, x, (((1,), (1,)), ((), ())),
                               preferred_element_type=jnp.float32)


def _manual_mv(slot_ref, xb, rows):
    n = xb.shape[1]
    nch = n // 128
    cols = []
    for g in range(rows // 8):
        prod = slot_ref[pl.ds(g * 8, 8), :] * xb
        parts = [prod[:, c * 128:(c + 1) * 128] for c in range(nch)]
        while len(parts) > 1:
            nxt = [parts[i] + parts[i + 1] for i in range(0, len(parts) - 1, 2)]
            if len(parts) % 2:
                nxt.append(parts[-1])
            parts = nxt
        cols.append(jnp.sum(parts[0], axis=1, keepdims=True))
    return jnp.concatenate(cols, axis=0)


def _mega_body(fs_ref, wih_ref, whh_ref, wout_ref, xw_ref, xcat_ref,
               y_ref, ring_ref, ring2_ref, sems, sems2):
    y_ref[...] = jnp.zeros_like(y_ref)
    xw = xw_ref[...]

    def wide_phase(src_ref, n_tiles, out_base):
        def issue(t, b):
            pltpu.make_async_copy(
                src_ref.at[pl.ds(t * 256, 256)], ring_ref.at[b],
                sems.at[b]).start()

        def wait_compute(t, b):
            pltpu.make_async_copy(
                src_ref.at[pl.ds(t * 256, 256)], ring_ref.at[b],
                sems.at[b]).wait()
            y = _manual_mv(ring_ref.at[b], xw, 256)
            y_ref[pl.ds(out_base + t * 256, 256), :] += y

        for b in range(_NWIDE):
            issue(b, b)
        n_groups = n_tiles // _NWIDE

        def group(g, _):
            for b in range(_NWIDE):
                t = g * _NWIDE + b
                wait_compute(t, b)

                @pl.when(g < n_groups - 1)
                def _pref():
                    issue(t + _NWIDE, b)
            return 0

        jax.lax.fori_loop(0, n_groups, group, 0)

    def narrow_phase(src_ref, n_tiles, tile_map):
        def issue(t, b):
            r0, c0, _, _ = tile_map(t)
            pltpu.make_async_copy(
                src_ref.at[pl.ds(r0, 256), pl.ds(c0, _NARROW)],
                ring2_ref.at[b], sems2.at[b]).start()

        def wait_compute(t, b):
            r0, c0, xsel, o0 = tile_map(t)
            pltpu.make_async_copy(
                src_ref.at[pl.ds(r0, 256), pl.ds(c0, _NARROW)],
                ring2_ref.at[b], sems2.at[b]).wait()
            xrow = xcat_ref[pl.ds(xsel * 8, 8), :]
            y = _manual_mv(ring2_ref.at[b], xrow, 256)
            y_ref[pl.ds(o0, 256), :] += y

        for b in range(_NBUF):
            issue(b, b)
        n_groups = n_tiles // _NBUF

        def group(g, _):
            for b in range(_NBUF):
                t = g * _NBUF + b
                wait_compute(t, b)

                @pl.when(g < n_groups - 1)
                def _pref():
                    issue(t + _NBUF, b)
            return 0

        jax.lax.fori_loop(0, n_groups, group, 0)

    wide_phase(fs_ref, 32, _ALPHA0)
    wide_phase(wih_ref, 24, _GI0)

    def whh_map(t):
        return t * 256, 0, 2, _GH0 + t * 256

    narrow_phase(whh_ref, 24, whh_map)

    def wout_map(t):
        i = t // 3
        j = t - 3 * i
        c0 = jnp.where(j == 2, 6144, j * _NARROW)
        return i * 256, c0, j, _PART0 + i * 256

    narrow_phase(wout_ref, 48, wout_map)


def _mega_matvec(fs, W_ih, W_hh, W_out, feature, h):
    xw = jnp.broadcast_to(feature.reshape(1, _WIDE), (8, _WIDE))
    xcat = jnp.concatenate(
        [jnp.broadcast_to(feature.reshape(2, 1, _NARROW), (2, 8, _NARROW)).reshape(16, _NARROW),
         jnp.broadcast_to(h.reshape(1, _NARROW), (8, _NARROW)),
         jnp.zeros((8, _NARROW), jnp.float32)], axis=0)
    return pl.pallas_call(
        _mega_body,
        grid=(1,),
        in_specs=[
            pl.BlockSpec(memory_space=pltpu.HBM),
            pl.BlockSpec(memory_space=pltpu.HBM),
            pl.BlockSpec(memory_space=pltpu.HBM),
            pl.BlockSpec(memory_space=pltpu.HBM),
            pl.BlockSpec(memory_space=pltpu.VMEM),
            pl.BlockSpec(memory_space=pltpu.VMEM),
        ],
        out_specs=pl.BlockSpec(memory_space=pltpu.VMEM),
        out_shape=jax.ShapeDtypeStruct((_MTOT, 1), jnp.float32),
        scratch_shapes=[
            pltpu.VMEM((_NWIDE, 256, _WIDE), jnp.float32),
            pltpu.VMEM((_NBUF, 256, _NARROW), jnp.float32),
            pltpu.SemaphoreType.DMA((_NWIDE,)),
            pltpu.SemaphoreType.DMA((_NBUF,)),
        ],
    )(fs, W_ih, W_hh, W_out, xw, xcat)


def _select_body(alpha_ref, elapsed_ref, hs_ref, attn_ref,
                 idx_ref, rows_ref, sem):
    alpha = alpha_ref[...]
    elapsed = elapsed_ref[...]
    rows_i = jax.lax.broadcasted_iota(jnp.int32, alpha.shape, 0)
    cols_i = jax.lax.broadcasted_iota(jnp.int32, alpha.shape, 1)
    flat_f = (rows_i * 1024 + cols_i).astype(jnp.float32)
    col64 = jax.lax.broadcasted_iota(jnp.int32, (1, _K), 1)

    def body(k, carry):
        masked, vals = carry
        m = jnp.max(masked)
        eq = masked == m
        idx_f = jnp.min(jnp.where(eq, flat_f, _POS_BIG))
        hit = flat_f == idx_f
        el = jnp.min(jnp.where(hit, elapsed, _POS_BIG))
        decayed = m * jnp.exp(_LN_EXP * el)
        vals = jnp.where(col64 == k, decayed, vals)
        idx_ref[k] = idx_f.astype(jnp.int32)
        masked = jnp.where(hit, _NEG_BIG, masked)
        return masked, vals

    _, vals = jax.lax.fori_loop(
        0, _K, body, (alpha, jnp.zeros((1, _K), jnp.float32)))

    vmax = jnp.max(vals)
    e = jnp.exp(vals - vmax)
    w = e / jnp.sum(e)

    for k in range(_K):
        pltpu.make_async_copy(
            hs_ref.at[pl.ds(idx_ref[k], 1)], rows_ref.at[pl.ds(k, 1)], sem
        ).start()
    for k in range(_K):
        pltpu.make_async_copy(
            hs_ref.at[pl.ds(idx_ref[k], 1)], rows_ref.at[pl.ds(k, 1)], sem
        ).wait()

    attn_ref[...] = jax.lax.dot_general(
        w, rows_ref[...], (((1,), (0,)), ((), ())),
        preferred_element_type=jnp.float32)


def _select_gather(alpha, elapsed, hs):
    h_dim = hs.shape[1]
    return pl.pallas_call(
        _select_body,
        in_specs=[
            pl.BlockSpec(memory_space=pltpu.VMEM),
            pl.BlockSpec(memory_space=pltpu.VMEM),
            pl.BlockSpec(memory_space=pltpu.HBM),
        ],
        out_specs=pl.BlockSpec(memory_space=pltpu.VMEM),
        out_shape=jax.ShapeDtypeStruct((1, h_dim), jnp.float32),
        scratch_shapes=[
            pltpu.SMEM((_K,), jnp.int32),
            pltpu.VMEM((_K, h_dim), jnp.float32),
            pltpu.SemaphoreType.DMA,
        ],
    )(alpha.reshape(8, 1024), elapsed.reshape(8, 1024), hs)


def _final_body(wmid_ref, attn_ref, partial_ref, be_ref, gi_ref, gh_ref,
                bih_ref, bhh_ref, h_ref, out_ref, hnew_ref):
    pred = partial_ref[...] + be_ref[...] + jax.lax.dot_general(
        attn_ref[...], wmid_ref[...], (((1,), (1,)), ((), ())),
        preferred_element_type=jnp.float32)
    m = jnp.max(pred)
    lse = jnp.log(jnp.sum(jnp.exp(pred - m))) + m
    out_ref[...] = pred - lse

    gi = gi_ref[...] + bih_ref[...]
    gh = gh_ref[...] + bhh_ref[...]
    hdim = h_ref.shape[1]
    i_r = gi[:, :hdim]
    i_z = gi[:, hdim:2 * hdim]
    i_n = gi[:, 2 * hdim:]
    h_r = gh[:, :hdim]
    h_z = gh[:, hdim:2 * hdim]
    h_n = gh[:, 2 * hdim:]
    r = jax.nn.sigmoid(i_r + h_r)
    z = jax.nn.sigmoid(i_z + h_z)
    n = jnp.tanh(i_n + r * h_n)
    hnew_ref[...] = (1.0 - z) * n + z * h_ref[...]


def kernel(feature, time, fs, hs, ts, W_ih, W_hh, b_ih, b_hh, W_out, b_out):
    feature = feature.astype(jnp.float32)
    L, in_dim = fs.shape
    h_dim = hs.shape[1]
    out_dim = W_out.shape[0]
    h = hs[-1]

    elapsed = jnp.float32(time) - ts

    w_last = jax.lax.slice(W_out, (0, in_dim + 2 * h_dim),
                           (out_dim, in_dim + 2 * h_dim + 1))
    bias_eff = (b_out + float(_K) * w_last.reshape(-1)).reshape(1, out_dim)

    y = _mega_matvec(fs, W_ih, W_hh, W_out, feature, h)
    alpha = y[_ALPHA0:_ALPHA0 + L]
    gi = y[_GI0:_GI0 + 3 * h_dim]
    gh = y[_GH0:_GH0 + 3 * h_dim]
    partial = y[_PART0:_PART0 + out_dim]

    attn = _select_gather(alpha.reshape(-1), elapsed, hs)

    output, h_new = pl.pallas_call(
        _final_body,
        grid=(1,),
        in_specs=[
            pl.BlockSpec((out_dim, h_dim), lambda i: (0, 2)),
            pl.BlockSpec(memory_space=pltpu.VMEM),
            pl.BlockSpec(memory_space=pltpu.VMEM),
            pl.BlockSpec(memory_space=pltpu.VMEM),
            pl.BlockSpec(memory_space=pltpu.VMEM),
            pl.BlockSpec(memory_space=pltpu.VMEM),
            pl.BlockSpec(memory_space=pltpu.VMEM),
            pl.BlockSpec(memory_space=pltpu.VMEM),
            pl.BlockSpec(memory_space=pltpu.VMEM),
        ],
        out_specs=[
            pl.BlockSpec(memory_space=pltpu.VMEM),
            pl.BlockSpec(memory_space=pltpu.VMEM),
        ],
        out_shape=[
            jax.ShapeDtypeStruct((1, out_dim), jnp.float32),
            jax.ShapeDtypeStruct((1, h_dim), jnp.float32),
        ],
    )(W_out, attn, partial.reshape(1, out_dim), bias_eff,
      gi.reshape(1, 3 * h_dim), gh.reshape(1, 3 * h_dim),
      b_ih.reshape(1, 3 * h_dim), b_hh.reshape(1, 3 * h_dim),
      h.reshape(1, h_dim))

    return output, h_new


# speedup vs baseline: 1.0926x; 1.0440x over previous
import math

import jax
import jax.numpy as jnp
from jax.experimental import pallas as pl
from jax.experimental.pallas import tpu as pltpu

_K = 64
_EXP = 0.999
_LN_EXP = math.log(_EXP)
_NEG_BIG = -3.0e38
_POS_BIG = 3.0e38

_NBUF = 8
_NWIDE = 6
_BM = 128
_WIDE = 4096
_NARROW = 2048

_ALPHA0 = 0
_GI0 = 8192
_GH0 = 8192 + 6144
_PART0 = 8192 + 6144 + 6144
_MTOT = 8192 + 6144 + 6144 + 4096


def _dot_nt(w, x):
    return jax.lax.dot_general(w, x, (((1,), (1,)), ((), ())),
                               preferred_element_type=jnp.float32)


def _manual_mv(slot_ref, xb, rows):
    n = xb.shape[1]
    nch = n // 128
    cols = []
    for g in range(rows // 8):
        prod = slot_ref[pl.ds(g * 8, 8), :] * xb
        parts = [prod[:, c * 128:(c + 1) * 128] for c in range(nch)]
        while len(parts) > 1:
            nxt = [parts[i] + parts[i + 1] for i in range(0, len(parts) - 1, 2)]
            if len(parts) % 2:
                nxt.append(parts[-1])
            parts = nxt
        cols.append(jnp.sum(parts[0], axis=1, keepdims=True))
    return jnp.concatenate(cols, axis=0)


def _mega_body(fs_ref, wih_ref, whh_ref, wout_ref, xw_ref, xcat_ref,
               y_ref, ring_ref, ring2_ref, sems, sems2):
    y_ref[...] = jnp.zeros_like(y_ref)
    xw = xw_ref[...]

    def wide_phase(src_ref, n_tiles, out_base):
        def issue(t, b):
            pltpu.make_async_copy(
                src_ref.at[pl.ds(t * 256, 256)], ring_ref.at[b],
                sems.at[b]).start()

        def wait_compute(t, b):
            pltpu.make_async_copy(
                src_ref.at[pl.ds(t * 256, 256)], ring_ref.at[b],
                sems.at[b]).wait()
            y = _manual_mv(ring_ref.at[b], xw, 256)
            y_ref[pl.ds(out_base + t * 256, 256), :] += y

        for b in range(_NWIDE):
            issue(b, b)
        n_groups = n_tiles // _NWIDE

        def group(g, _):
            for b in range(_NWIDE):
                t = g * _NWIDE + b
                wait_compute(t, b)

                @pl.when(g < n_groups - 1)
                def _pref():
                    issue(t + _NWIDE, b)
            return 0

        jax.lax.fori_loop(0, n_groups, group, 0)

    def narrow_phase(src_ref, n_tiles, tile_map):
        def issue(t, b):
            r0, c0, _, _ = tile_map(t)
            pltpu.make_async_copy(
                src_ref.at[pl.ds(r0, 256), pl.ds(c0, _NARROW)],
                ring2_ref.at[b], sems2.at[b]).start()

        def wait_compute(t, b):
            r0, c0, xsel, o0 = tile_map(t)
            pltpu.make_async_copy(
                src_ref.at[pl.ds(r0, 256), pl.ds(c0, _NARROW)],
                ring2_ref.at[b], sems2.at[b]).wait()
            xrow = xcat_ref[pl.ds(xsel * 8, 8), :]
            y = _manual_mv(ring2_ref.at[b], xrow, 256)
            y_ref[pl.ds(o0, 256), :] += y

        for b in range(_NBUF):
            issue(b, b)
        n_groups = n_tiles // _NBUF

        def group(g, _):
            for b in range(_NBUF):
                t = g * _NBUF + b
                wait_compute(t, b)

                @pl.when(g < n_groups - 1)
                def _pref():
                    issue(t + _NBUF, b)
            return 0

        jax.lax.fori_loop(0, n_groups, group, 0)

    wide_phase(fs_ref, 32, _ALPHA0)
    wide_phase(wih_ref, 24, _GI0)

    def whh_map(t):
        return t * 256, 0, 2, _GH0 + t * 256

    narrow_phase(whh_ref, 24, whh_map)

    def wout_map(t):
        i = t // 3
        j = t - 3 * i
        c0 = jnp.where(j == 2, 6144, j * _NARROW)
        return i * 256, c0, j, _PART0 + i * 256

    narrow_phase(wout_ref, 48, wout_map)


def _mega_matvec(fs, W_ih, W_hh, W_out, feature, h):
    xw = jnp.broadcast_to(feature.reshape(1, _WIDE), (8, _WIDE))
    xcat = jnp.concatenate(
        [jnp.broadcast_to(feature.reshape(2, 1, _NARROW), (2, 8, _NARROW)).reshape(16, _NARROW),
         jnp.broadcast_to(h.reshape(1, _NARROW), (8, _NARROW)),
         jnp.zeros((8, _NARROW), jnp.float32)], axis=0)
    return pl.pallas_call(
        _mega_body,
        grid=(1,),
        in_specs=[
            pl.BlockSpec(memory_space=pltpu.HBM),
            pl.BlockSpec(memory_space=pltpu.HBM),
            pl.BlockSpec(memory_space=pltpu.HBM),
            pl.BlockSpec(memory_space=pltpu.HBM),
            pl.BlockSpec(memory_space=pltpu.VMEM),
            pl.BlockSpec(memory_space=pltpu.VMEM),
        ],
        out_specs=pl.BlockSpec(memory_space=pltpu.VMEM),
        out_shape=jax.ShapeDtypeStruct((_MTOT, 1), jnp.float32),
        scratch_shapes=[
            pltpu.VMEM((_NWIDE, 256, _WIDE), jnp.float32),
            pltpu.VMEM((_NBUF, 256, _NARROW), jnp.float32),
            pltpu.SemaphoreType.DMA((_NWIDE,)),
            pltpu.SemaphoreType.DMA((_NBUF,)),
        ],
    )(fs, W_ih, W_hh, W_out, xw, xcat)


def _lane_shift_cumsum(x):
    n = x.shape[1]
    sh = 1
    while sh < n:
        x = x + jnp.concatenate(
            [jnp.zeros((x.shape[0], sh), x.dtype), x[:, :-sh]], axis=1)
        sh *= 2
    return x


def _row_shift_cumsum(x):
    n = x.shape[0]
    sh = 1
    while sh < n:
        x = x + jnp.concatenate(
            [jnp.zeros((sh, x.shape[1]), x.dtype), x[:-sh, :]], axis=0)
        sh *= 2
    return x


def _select_body(alpha_ref, elapsed_ref, idx_ref, w_ref):
    alpha = alpha_ref[...]
    elapsed = elapsed_ref[...]

    ai = jax.lax.bitcast_convert_type(alpha, jnp.int32)
    key = ai ^ (jax.lax.shift_right_arithmetic(ai, 31) & jnp.int32(0x7FFFFFFF))

    n_pos = jnp.sum((key >= 0).astype(jnp.int32))
    kneed = jnp.where(n_pos >= _K, _K, _K - n_pos)
    pos_i = (key >= 0).astype(jnp.int32)
    class_mask = pos_i == jnp.where(n_pos >= _K, 1, 0)
    v = key & jnp.int32(0x7FFFFFFF)

    def bit_body(i, P):
        T = P | jax.lax.shift_left(jnp.int32(1), 30 - i)
        c = jnp.sum(jnp.where(class_mask & (v >= T), 1, 0).astype(jnp.int32))
        return jnp.where(c >= kneed, T, P)

    P = jax.lax.fori_loop(0, 31, bit_body, jnp.int32(0))
    key_t = jnp.where(n_pos >= _K, P, P | jnp.int32(-2147483648))

    in_gt = key > key_t
    n_gt = jnp.sum(in_gt.astype(jnp.int32))
    need_ties = _K - n_gt
    tie = key == key_t
    tie_i = tie.astype(jnp.int32)
    lane_c = _lane_shift_cumsum(tie_i)
    row_tot = lane_c[:, -1:]
    row_pre = _row_shift_cumsum(row_tot) - row_tot
    rank_tie = row_pre + lane_c - tie_i
    tie_sel = tie & (rank_tie < need_ties)
    selected = in_gt | tie_sel

    dec = alpha * jnp.exp(_LN_EXP * elapsed)
    dsel = jnp.where(selected, dec, _NEG_BIG)
    m64 = jnp.max(dsel)
    e = jnp.exp(dsel - m64)
    s = jnp.sum(e)
    wfull = e / s

    sel_i = selected.astype(jnp.int32)
    lane_s = _lane_shift_cumsum(sel_i)
    rtot = lane_s[:, -1:]
    rpre = _row_shift_cumsum(rtot) - rtot
    rank = rpre + lane_s - sel_i

    cols_i = jax.lax.broadcasted_iota(jnp.int32, alpha.shape, 1)
    iota64c = jax.lax.broadcasted_iota(jnp.int32, (_K, 1), 0)
    widx = jnp.zeros((_K, 1), jnp.float32)
    iidx = jnp.zeros((_K, 1), jnp.int32)
    for r in range(8):
        oh = (rank[r:r + 1, :] == iota64c) & selected[r:r + 1, :]
        widx = widx + jnp.sum(oh.astype(jnp.float32) * wfull[r:r + 1, :],
                              axis=1, keepdims=True)
        iidx = iidx + jnp.sum(
            jnp.where(oh, cols_i[r:r + 1, :] + r * 1024, 0),
            axis=1, keepdims=True)
    idx_ref[...] = iidx
    w_ref[...] = widx


def _select(alpha, elapsed):
    return pl.pallas_call(
        _select_body,
        in_specs=[
            pl.BlockSpec(memory_space=pltpu.VMEM),
            pl.BlockSpec(memory_space=pltpu.VMEM),
        ],
        out_specs=[
            pl.BlockSpec(memory_space=pltpu.VMEM),
            pl.BlockSpec(memory_space=pltpu.VMEM),
        ],
        out_shape=[
            jax.ShapeDtypeStruct((_K, 1), jnp.int32),
            jax.ShapeDtypeStruct((_K, 1), jnp.float32),
        ],
    )(alpha.reshape(8, 1024), elapsed.reshape(8, 1024))


def _head_body(idx_ref, w_ref, hs_ref, wout_ref, part_ref, be_ref,
               gi_ref, gh_ref, bih_ref, bhh_ref, h_ref,
               out_ref, hnew_ref, rows_ref, ring_ref, rsems, gsem):
    for k in range(_K):
        pltpu.make_async_copy(
            hs_ref.at[pl.ds(idx_ref[k, 0], 1)], rows_ref.at[pl.ds(k, 1)],
            gsem).start()

    def wissue(t, b):
        pltpu.make_async_copy(
            wout_ref.at[pl.ds(t * 256, 256), pl.ds(4096, _NARROW)],
            ring_ref.at[b], rsems.at[b]).start()

    def wwait(t, b):
        pltpu.make_async_copy(
            wout_ref.at[pl.ds(t * 256, 256), pl.ds(4096, _NARROW)],
            ring_ref.at[b], rsems.at[b]).wait()

    for b in range(_NBUF):
        wissue(b, b)

    for k in range(_K):
        pltpu.make_async_copy(
            hs_ref.at[pl.ds(idx_ref[k, 0], 1)], rows_ref.at[pl.ds(k, 1)],
            gsem).wait()
    attn = jax.lax.dot_general(
        w_ref[...], rows_ref[...], (((0,), (0,)), ((), ())),
        preferred_element_type=jnp.float32)
    attnb = jnp.broadcast_to(attn, (8, _NARROW))

    segs = []
    for t in range(16):
        b = t % _NBUF
        wwait(t, b)
        segs.append(_manual_mv(ring_ref.at[b], attnb, 256))
        if t + _NBUF < 16:
            wissue(t + _NBUF, b)
    pred = part_ref[...] + jnp.concatenate(segs, axis=0)
    pred = pred + be_ref[...]
    m = jnp.max(pred)
    lse = jnp.log(jnp.sum(jnp.exp(pred - m))) + m
    out_ref[...] = pred - lse

    gi = gi_ref[...] + bih_ref[...]
    gh = gh_ref[...] + bhh_ref[...]
    hdim = h_ref.shape[1]
    i_r = gi[:, :hdim]
    i_z = gi[:, hdim:2 * hdim]
    i_n = gi[:, 2 * hdim:]
    h_r = gh[:, :hdim]
    h_z = gh[:, hdim:2 * hdim]
    h_n = gh[:, 2 * hdim:]
    r = jax.nn.sigmoid(i_r + h_r)
    z = jax.nn.sigmoid(i_z + h_z)
    n = jnp.tanh(i_n + r * h_n)
    hnew_ref[...] = (1.0 - z) * n + z * h_ref[...]


def kernel(feature, time, fs, hs, ts, W_ih, W_hh, b_ih, b_hh, W_out, b_out):
    feature = feature.astype(jnp.float32)
    L, in_dim = fs.shape
    h_dim = hs.shape[1]
    out_dim = W_out.shape[0]
    h = hs[-1]

    elapsed = jnp.float32(time) - ts

    w_last = jax.lax.slice(W_out, (0, in_dim + 2 * h_dim),
                           (out_dim, in_dim + 2 * h_dim + 1))
    bias_eff = (b_out + float(_K) * w_last.reshape(-1)).reshape(1, out_dim)

    y = _mega_matvec(fs, W_ih, W_hh, W_out, feature, h)
    alpha = y[_ALPHA0:_ALPHA0 + L]
    gi = y[_GI0:_GI0 + 3 * h_dim]
    gh = y[_GH0:_GH0 + 3 * h_dim]
    partial = y[_PART0:_PART0 + out_dim]

    idx, w = _select(alpha.reshape(-1), elapsed)

    output, h_new = pl.pallas_call(
        _head_body,
        grid=(1,),
        in_specs=[
            pl.BlockSpec(memory_space=pltpu.SMEM),
            pl.BlockSpec(memory_space=pltpu.VMEM),
            pl.BlockSpec(memory_space=pltpu.HBM),
            pl.BlockSpec(memory_space=pltpu.HBM),
            pl.BlockSpec(memory_space=pltpu.VMEM),
            pl.BlockSpec(memory_space=pltpu.VMEM),
            pl.BlockSpec(memory_space=pltpu.VMEM),
            pl.BlockSpec(memory_space=pltpu.VMEM),
            pl.BlockSpec(memory_space=pltpu.VMEM),
            pl.BlockSpec(memory_space=pltpu.VMEM),
            pl.BlockSpec(memory_space=pltpu.VMEM),
        ],
        out_specs=[
            pl.BlockSpec(memory_space=pltpu.VMEM),
            pl.BlockSpec(memory_space=pltpu.VMEM),
        ],
        out_shape=[
            jax.ShapeDtypeStruct((out_dim, 1), jnp.float32),
            jax.ShapeDtypeStruct((1, h_dim), jnp.float32),
        ],
        scratch_shapes=[
            pltpu.VMEM((_K, h_dim), jnp.float32),
            pltpu.VMEM((_NBUF, 256, _NARROW), jnp.float32),
            pltpu.SemaphoreType.DMA((_NBUF,)),
            pltpu.SemaphoreType.DMA,
        ],
    )(idx, w, hs, W_out, partial, bias_eff.reshape(out_dim, 1),
      gi.reshape(1, 3 * h_dim), gh.reshape(1, 3 * h_dim),
      b_ih.reshape(1, 3 * h_dim), b_hh.reshape(1, 3 * h_dim),
      h.reshape(1, h_dim))

    return output.reshape(1, out_dim), h_new
